# Initial kernel scaffold; baseline (speedup 1.0000x reference)
import jax
import jax.numpy as jnp
from jax.experimental import pallas as pl


def kernel(x, edge_index, batch, Wrel1, brel1, Wroot1, p1, Wrel2, brel2, Wroot2, p2, Wrel3, brel3, Wroot3, p3, W1, b1, W2, b2, W3, b3):
    raise NotImplementedError("write your pallas kernel here")


# speedup vs baseline: 4.5144x; 4.5144x over previous
import functools

import jax
import jax.numpy as jnp
from jax import lax
from jax.experimental import pallas as pl
from jax.experimental.pallas import tpu as pltpu
from jax.experimental.pallas import tpu_sc as plsc

N = 50000
NPAD = 50176
E = 800000
H = 128
NW = 32
EPW = 25088
EPAD = NW * EPW
CH = 128
EPS = EPAD // 16
NCHUNK = EPS // CH
RN = 12544
TR = 784
SPM_ROWS = RN + 8
RB = 6272
NB = NPAD // RB

K1 = 25000
K2 = 20000
K3 = 16000

def _mm(a, b):
    return jnp.dot(a, b, preferred_element_type=jnp.float32,
                   precision=lax.Precision.HIGHEST)


def _sc_mesh():
    return plsc.VectorSubcoreMesh(
        core_axis_name="c", subcore_axis_name="s", num_cores=2,
        num_subcores=16)


def _sc_agg128(hsrc, srcp, dstp, zeros128, out, idx_v, dst_v, rows_v, shared,
               sem):
    c = lax.axis_index("c")
    s = lax.axis_index("s")
    base = s * EPS

    for p in range(2):
        lo = (2 * p + c) * RN
        pltpu.sync_copy(zeros128, shared.at[pl.ds(s * TR, TR)])
        @pl.when(s == 0)
        def _():
            pltpu.sync_copy(zeros128.at[pl.ds(0, 8)], shared.at[pl.ds(RN, 8)])
        plsc.subcore_barrier()

        def _step(k, _):
            off = base + k * CH
            pltpu.sync_copy(srcp.at[pl.ds(off, CH)], idx_v)
            pltpu.sync_copy(dstp.at[pl.ds(off, CH)], dst_v)
            for j in range(CH // 16):
                dv = dst_v[pl.ds(j * 16, 16)] - lo
                oob = (dv < 0) | (dv >= RN)
                dst_v[pl.ds(j * 16, 16)] = jnp.where(oob, RN, dv)
            pltpu.async_copy(hsrc.at[idx_v], rows_v, sem).wait()
            pltpu.sync_copy(rows_v, shared.at[dst_v], add=True)
            return 0
        lax.fori_loop(0, NCHUNK, _step, 0)
        plsc.subcore_barrier()
        pltpu.sync_copy(shared.at[pl.ds(s * TR, TR)],
                        out.at[pl.ds(lo + s * TR, TR)])
        plsc.subcore_barrier()


def _run_sc_agg128(hsrc, srcp, dstp):
    f = pl.kernel(
        _sc_agg128,
        out_type=jax.ShapeDtypeStruct((NPAD, H), jnp.float32),
        mesh=_sc_mesh(),
        scratch_types=[
            pltpu.VMEM((CH,), jnp.int32),
            pltpu.VMEM((CH,), jnp.int32),
            pltpu.VMEM((CH, H), jnp.float32),
            pltpu.VMEM_SHARED((SPM_ROWS, H), jnp.float32),
            pltpu.SemaphoreType.DMA,
        ],
    )
    return f(hsrc, srcp, dstp, jnp.zeros((TR, H), jnp.float32))


def _score(h, pv):
    s = lax.dot_general(h, pv, (((1,), (1,)), ((), ())),
                        preferred_element_type=jnp.float32,
                        precision=lax.Precision.HIGHEST)
    return s / jnp.sqrt(jnp.sum(pv * pv))


def _dense1_body(aggr, xb, wrel, wroot, brel, pvec, alive, h_out, s_out):
    h = _mm(aggr[...], wrel[...]) + brel[...] + _mm(xb[...], wroot[...])
    h = jnp.maximum(h, 0.0)
    s = _score(h, pvec[...])
    h_out[...] = h
    s_out[...] = jnp.where(alive[...] > 0, s, -jnp.inf)


def _run_dense1(aggr, xpad, wrel, wroot, brel, pvec, alive):
    blk = lambda w: pl.BlockSpec((RB, w), lambda i: (i, 0))
    full = lambda a, b: pl.BlockSpec((a, b), lambda i: (0, 0))
    return pl.pallas_call(
        _dense1_body,
        grid=(NB,),
        in_specs=[blk(H), blk(16), full(H, H), full(16, H), full(1, H),
                  full(1, H), blk(1)],
        out_specs=[blk(H), blk(1)],
        out_shape=[jax.ShapeDtypeStruct((NPAD, H), jnp.float32),
                   jax.ShapeDtypeStruct((NPAD, 1), jnp.float32)],
    )(aggr, xpad, wrel, wroot, brel, pvec, alive)


def _dense_body(aggr, hb, wrel, wroot, brel, pvec, alive, h_out, s_out):
    h = _mm(aggr[...], wrel[...]) + brel[...] + _mm(hb[...], wroot[...])
    h = jnp.maximum(h, 0.0)
    s = _score(h, pvec[...])
    h_out[...] = h
    s_out[...] = jnp.where(alive[...] > 0, s, -jnp.inf)


def _run_dense(aggr, hprev, wrel, wroot, brel, pvec, alive):
    blk = lambda w: pl.BlockSpec((RB, w), lambda i: (i, 0))
    full = lambda a, b: pl.BlockSpec((a, b), lambda i: (0, 0))
    return pl.pallas_call(
        _dense_body,
        grid=(NB,),
        in_specs=[blk(H), blk(H), full(H, H), full(H, H),
                  full(1, H), full(1, H), blk(1)],
        out_specs=[blk(H), blk(1)],
        out_shape=[jax.ShapeDtypeStruct((NPAD, H), jnp.float32),
                   jax.ShapeDtypeStruct((NPAD, 1), jnp.float32)],
    )(aggr, hprev, wrel, wroot, brel, pvec, alive)


SROWS = NPAD // 128


def _thresh_body(k, s_ref, alive_ref):
    bits = lax.bitcast_convert_type(s_ref[...], jnp.int32)
    u = jnp.where(bits < 0, bits ^ jnp.int32(0x7FFFFFFF), bits)
    kf = jnp.float32(k)

    cnt0 = jnp.sum((u >= 0).astype(jnp.float32))
    t0 = jnp.where(cnt0 >= kf, jnp.int32(0), jnp.int32(-2147483648))

    def _bit(j, t):
        cand = t + lax.shift_left(jnp.int32(1), jnp.int32(30) - j)
        cnt = jnp.sum((u >= cand).astype(jnp.float32))
        return jnp.where(cnt >= kf, cand, t)
    t = lax.fori_loop(0, 31, _bit, t0)

    gt = u > t
    eq = u == t
    c_gt = jnp.sum(gt.astype(jnp.float32))
    need = kf - c_gt

    eqf = eq.astype(jnp.float32)
    ci = lax.broadcasted_iota(jnp.int32, (128, 128), 0)
    cj = lax.broadcasted_iota(jnp.int32, (128, 128), 1)
    lower_incl = (ci <= cj).astype(jnp.float32)
    p_incl = _mm(eqf, lower_incl)
    rowtot = p_incl[:, 127:128]
    ri = lax.broadcasted_iota(jnp.int32, (SROWS, SROWS), 0)
    rj = lax.broadcasted_iota(jnp.int32, (SROWS, SROWS), 1)
    strict = (rj < ri).astype(jnp.float32)
    rowoff = _mm(strict, rowtot)
    g_excl = rowoff + (p_incl - eqf)
    alive = gt | (eq & (g_excl < need))
    alive_ref[...] = alive.astype(jnp.float32)


def _run_thresh(s_r, k):
    return pl.pallas_call(
        functools.partial(_thresh_body, k),
        out_shape=jax.ShapeDtypeStruct((SROWS, 128), jnp.float32),
    )(s_r)


def _pool_body(k, h_ref, s_ref, alive_ref, hs_ref, pooled_ref, acc):
    i = pl.program_id(0)

    @pl.when(i == 0)
    def _():
        acc[0:1, :] = jnp.full((1, H), -jnp.inf, jnp.float32)
        acc[1:2, :] = jnp.zeros((1, H), jnp.float32)

    scale = jnp.tanh(s_ref[...]) * alive_ref[...]
    hs = h_ref[...] * scale
    hs_ref[...] = hs
    mx = jnp.max(jnp.where(alive_ref[...] > 0, hs, -jnp.inf),
                 axis=0, keepdims=True)
    sm = jnp.sum(hs, axis=0, keepdims=True)
    acc[0:1, :] = jnp.maximum(acc[0:1, :], mx)
    acc[1:2, :] = acc[1:2, :] + sm

    @pl.when(i == NB - 1)
    def _():
        pooled_ref[0:1, :] = acc[0:1, :]
        pooled_ref[1:2, :] = acc[1:2, :] * jnp.float32(1.0 / k)


def _run_pool(h, s, alive, k):
    blk = lambda w: pl.BlockSpec((RB, w), lambda i: (i, 0))
    return pl.pallas_call(
        functools.partial(_pool_body, k),
        grid=(NB,),
        in_specs=[blk(H), blk(1), blk(1)],
        out_specs=[blk(H), pl.BlockSpec((2, H), lambda i: (0, 0))],
        out_shape=[jax.ShapeDtypeStruct((NPAD, H), jnp.float32),
                   jax.ShapeDtypeStruct((2, H), jnp.float32)],
        scratch_shapes=[pltpu.VMEM((2, H), jnp.float32)],
    )(h, s, alive)


def _mlp_body(z, w1, b1, w2, b2, w3, b3, out):
    o = jnp.maximum(_mm(z[...], w1[...]) + b1[...], 0.0)
    o = jnp.maximum(_mm(o, w2[...]) + b2[...], 0.0)
    out[...] = _mm(o, w3[...]) + b3[...]


def _run_mlp(z, w1, b1, w2, b2, w3, b3):
    return pl.pallas_call(
        _mlp_body,
        out_shape=jax.ShapeDtypeStruct((1, 1), jnp.float32),
    )(z, w1, b1.reshape(1, -1), w2, b2.reshape(1, -1), w3, b3.reshape(1, -1))


def kernel(x, edge_index, batch, Wrel1, brel1, Wroot1, p1, Wrel2, brel2,
           Wroot2, p2, Wrel3, brel3, Wroot3, p3, W1, b1, W2, b2, W3, b3):
    src = edge_index[0]
    dst = edge_index[1]
    pad = EPAD - E
    srcp = jnp.concatenate([src, jnp.zeros((pad,), jnp.int32)])
    dstp = jnp.concatenate([dst, jnp.full((pad,), N, jnp.int32)])

    xpad = jnp.zeros((NPAD, 16), jnp.float32).at[:N, :11].set(x)
    xwide = jnp.zeros((NPAD, H), jnp.float32).at[:N, :11].set(x)
    wrel1p = jnp.zeros((H, H), jnp.float32).at[:11].set(Wrel1)
    wroot1p = jnp.zeros((16, H), jnp.float32).at[:11].set(Wroot1)
    alive0 = jnp.concatenate(
        [jnp.ones((N,), jnp.float32), jnp.zeros((NPAD - N,), jnp.float32)]
    ).reshape(NPAD, 1)

    aggr = _run_sc_agg128(xwide, srcp, dstp)
    h, s = _run_dense1(aggr, xpad, wrel1p, wroot1p, brel1.reshape(1, H),
                       p1.reshape(1, H), alive0)
    alive = _run_thresh(s.reshape(SROWS, 128), K1).reshape(NPAD, 1)
    h, x1 = _run_pool(h, s, alive, K1)

    aggr = _run_sc_agg128(h, srcp, dstp)
    h, s = _run_dense(aggr, h, Wrel2, Wroot2, brel2.reshape(1, H),
                      p2.reshape(1, H), alive)
    alive = _run_thresh(s.reshape(SROWS, 128), K2).reshape(NPAD, 1)
    h, x2 = _run_pool(h, s, alive, K2)

    aggr = _run_sc_agg128(h, srcp, dstp)
    h, s = _run_dense(aggr, h, Wrel3, Wroot3, brel3.reshape(1, H),
                      p3.reshape(1, H), alive)
    alive = _run_thresh(s.reshape(SROWS, 128), K3).reshape(NPAD, 1)
    h, x3 = _run_pool(h, s, alive, K3)

    z = (x1 + x2 + x3).reshape(1, 2 * H)
    return _run_mlp(z, W1, b1, W2, b2, W3, b3)
